# baseline (speedup 1.0000x reference)
import functools

import jax
import jax.numpy as jnp
from jax import lax
from jax.experimental import pallas as pl
from jax.experimental.pallas import tpu as pltpu
from jax.experimental.pallas import tpu_sc as plsc

VOCAB = 1000000
D_MODEL = 64
BATCH = 4096
HIST = 200

_info = plsc.get_sparse_core_info()
NC, NS = _info.num_cores, _info.num_subcores
NW = NC * NS

N_TCOL = 7813
VPAD = N_TCOL * 128
A_FULL = 7808
A_PER_W = A_FULL // NW
A_EXTRA = 7812 - A_FULL

NBLK = BATCH // 128
HGRP = HIST // 8
B_BLOCKS = HGRP * NBLK
B_BLK_PER_W = B_BLOCKS // NW

mesh = plsc.VectorSubcoreMesh(core_axis_name="c", subcore_axis_name="s")
_params = pltpu.CompilerParams(
    use_tc_tiling_on_sc=True, needs_layout_passes=False)

_IOTA16 = None


def _transpose_block(src, dst, n_rows, n_cols):
    iota = lax.iota(jnp.int32, 16)

    def row(r):
        rv = jnp.full((16,), r, jnp.int32)
        for cg in range(n_cols // 16):
            vals = plsc.load_gather(src, [iota + cg * 16, rv])
            dst[r, pl.ds(cg * 16, 16)] = vals

    def body(i, carry):
        for u in range(4):
            row(i * 4 + u)
        return carry

    lax.fori_loop(0, n_rows // 4, body, 0)


@functools.partial(
    pl.kernel,
    mesh=mesh,
    out_type=jax.ShapeDtypeStruct((VPAD, 128), jnp.float32),
    scratch_types=[
        pltpu.VMEM((2, D_MODEL, 128), jnp.float32),
        pltpu.VMEM((2, 128, 128), jnp.float32),
        [pltpu.SemaphoreType.DMA] * 2,
        [pltpu.SemaphoreType.DMA] * 2,
    ],
    compiler_params=_params,
)
def _table_transpose(tt_hbm, tail_hbm, scr_hbm, in_v, out_v, sem_i, sem_o):
    wid = lax.axis_index("s") * NC + lax.axis_index("c")
    c0w = wid * A_PER_W

    def _off(c):
        return pl.multiple_of(c * 128, 128)

    def start_in(c, b):
        pltpu.async_copy(
            tt_hbm.at[:, pl.ds(_off(c), 128)], in_v.at[b], sem_i[b])

    def wait_in(c, b):
        pltpu.make_async_copy(
            tt_hbm.at[:, pl.ds(_off(c), 128)], in_v.at[b], sem_i[b]).wait()

    def start_out(c, b):
        pltpu.async_copy(
            out_v.at[b], scr_hbm.at[pl.ds(_off(c), 128)], sem_o[b])

    def wait_out(c, b):
        pltpu.make_async_copy(
            out_v.at[b], scr_hbm.at[pl.ds(_off(c), 128)], sem_o[b]).wait()

    start_in(c0w, 0)

    def body(k, carry):
        for b in range(2):
            j = k * 2 + b
            c = c0w + j
            wait_in(c, b)
            start_in(c + 1, 1 - b)

            @pl.when(k > 0)
            def _():
                wait_out(c - 2, b)

            _transpose_block(in_v.at[b], out_v.at[b], 128, D_MODEL)
            start_out(c, b)
        return carry

    lax.fori_loop(0, A_PER_W // 2, body, 0)
    wait_in(c0w + A_PER_W, 0)
    wait_out(c0w + A_PER_W - 2, 0)
    wait_out(c0w + A_PER_W - 1, 1)

    @pl.when(wid < A_EXTRA)
    def _():
        c = A_FULL + wid
        pltpu.sync_copy(tt_hbm.at[:, pl.ds(c * 128, 128)], in_v.at[0])
        _transpose_block(in_v.at[0], out_v.at[0], 128, D_MODEL)
        pltpu.sync_copy(out_v.at[0], scr_hbm.at[pl.ds(c * 128, 128)])

    @pl.when(wid == A_EXTRA)
    def _():
        pltpu.sync_copy(tail_hbm, out_v.at[0])
        pltpu.sync_copy(out_v.at[0], scr_hbm.at[pl.ds(7812 * 128, 128)])


@functools.partial(
    pl.kernel,
    mesh=mesh,
    out_type=jax.ShapeDtypeStruct((HIST, D_MODEL, BATCH), jnp.float32),
    scratch_types=[
        pltpu.VMEM((2, 8, 128), jnp.int32),
        pltpu.VMEM((2, 128, 128), jnp.float32),
        pltpu.VMEM((2, D_MODEL, 128), jnp.float32),
        [pltpu.SemaphoreType.DMA] * 2,
        [pltpu.SemaphoreType.DMA] * 2,
        [pltpu.SemaphoreType.DMA] * 2,
    ],
    compiler_params=_params,
)
def _gather_transpose(xt_hbm, scr_hbm, out_hbm, idx_v, g_v, t_v,
                      sem_x, sem_g, sem_o):
    wid = lax.axis_index("s") * NC + lax.axis_index("c")
    n0 = wid * B_BLK_PER_W

    def hb(n):
        h0 = pl.multiple_of(lax.shift_left(lax.shift_right_logical(n, 5), 3), 8)
        b0 = pl.multiple_of(lax.shift_left(n & 31, 7), 128)
        return h0, b0

    def start_idx(n, u):
        h0, b0 = hb(n)
        pltpu.async_copy(
            xt_hbm.at[pl.ds(h0, 8), pl.ds(b0, 128)], idx_v.at[u], sem_x[u])

    def wait_idx(n, u):
        h0, b0 = hb(n)
        pltpu.make_async_copy(
            xt_hbm.at[pl.ds(h0, 8), pl.ds(b0, 128)], idx_v.at[u],
            sem_x[u]).wait()

    def start_gather(u, s, b):
        pltpu.async_copy(scr_hbm.at[idx_v.at[u].at[s]], g_v.at[b], sem_g[b])

    def wait_gather(u, s, b):
        pltpu.make_async_copy(
            scr_hbm.at[idx_v.at[u].at[s]], g_v.at[b], sem_g[b]).wait()

    def start_out(n, s, b):
        h0, b0 = hb(n)
        pltpu.async_copy(
            t_v.at[b], out_hbm.at[h0 + s, :, pl.ds(b0, 128)], sem_o[b])

    def wait_out(n, s, b):
        h0, b0 = hb(n)
        pltpu.make_async_copy(
            t_v.at[b], out_hbm.at[h0 + s, :, pl.ds(b0, 128)], sem_o[b]).wait()

    def block(n, u, prefetch_next):
        start_gather(u, 0, 0)
        if prefetch_next:
            start_idx(n + 1, 1 - u)
        for s in range(8):
            b = s % 2
            if s < 7:
                start_gather(u, s + 1, 1 - b)
            wait_gather(u, s, b)
            if s >= 2:
                wait_out(n, s - 2, b)
            _transpose_block(g_v.at[b], t_v.at[b], D_MODEL, 128)
            start_out(n, s, b)
        wait_out(n, 6, 0)
        wait_out(n, 7, 1)

    start_idx(n0, 0)

    def body(k, carry):
        for u in range(2):
            n = n0 + k * 2 + u
            wait_idx(n, u)
            block(n, u, prefetch_next=True)
        return carry

    lax.fori_loop(0, B_BLK_PER_W // 2, body, 0)
    n = n0 + B_BLK_PER_W - 1
    wait_idx(n, 0)
    block(n, 0, prefetch_next=False)


def kernel(x, table):
    xt = x.T
    tt = table.T
    tail = jnp.pad(table[VOCAB - 64:], ((0, 64), (0, 64)))
    scratch = _table_transpose(tt, tail)
    out_t = _gather_transpose(xt, scratch)
    return jnp.transpose(out_t, (2, 0, 1))


# speedup vs baseline: 2.2000x; 2.2000x over previous
import functools

import jax
import jax.numpy as jnp
from jax import lax
from jax.experimental import pallas as pl
from jax.experimental.pallas import tpu as pltpu
from jax.experimental.pallas import tpu_sc as plsc

VOCAB = 1000000
D_MODEL = 64
BATCH = 4096
HIST = 200

_info = plsc.get_sparse_core_info()
NC, NS = _info.num_cores, _info.num_subcores
NW = NC * NS

N_TCOL = 7813
VPAD = N_TCOL * 128
A_FULL = 7808
A_PER_W = A_FULL // NW
A_EXTRA = 7812 - A_FULL

NBLK = BATCH // 128
HGRP = HIST // 8
B_BLOCKS = HGRP * NBLK
B_BLK_PER_W = B_BLOCKS // NW

mesh = plsc.VectorSubcoreMesh(core_axis_name="c", subcore_axis_name="s")
_params = pltpu.CompilerParams(
    use_tc_tiling_on_sc=True, needs_layout_passes=False)

def _perms():
    iota = lax.iota(jnp.int32, 16)
    return iota, [(iota + k) & 15 for k in range(16)]


def _transpose_dmajor_to_vmajor(src, dst, iota, perms, n_vgrp):
    drows = [iota + dg * 16 for dg in range(D_MODEL // 16)]

    def body(vg, carry):
        v0 = vg * 16
        cols = [v0 + p for p in perms]
        for dg in range(D_MODEL // 16):
            for k in range(16):
                vals = plsc.load_gather(src, [drows[dg], cols[k]])
                plsc.store_scatter(dst, [cols[k], drows[dg]], vals)
        return carry

    lax.fori_loop(0, n_vgrp, body, 0)


def _transpose_vmajor_to_dmajor(src, dst, iota, perms):
    def body(bg, carry):
        brows = bg * 16 + iota
        for dg in range(D_MODEL // 16):
            for k in range(16):
                dcol = perms[k] + dg * 16
                vals = plsc.load_gather(src, [brows, dcol])
                plsc.store_scatter(dst, [dcol, brows], vals)
        return carry

    lax.fori_loop(0, 8, body, 0)


@functools.partial(
    pl.kernel,
    mesh=mesh,
    out_type=jax.ShapeDtypeStruct((VPAD, 128), jnp.float32),
    scratch_types=[
        pltpu.VMEM((2, D_MODEL, 128), jnp.float32),
        pltpu.VMEM((2, 128, 128), jnp.float32),
        [pltpu.SemaphoreType.DMA] * 2,
        [pltpu.SemaphoreType.DMA] * 2,
    ],
    compiler_params=_params,
)
def _table_transpose(tt_hbm, tail_hbm, scr_hbm, in_v, out_v, sem_i, sem_o):
    wid = lax.axis_index("s") * NC + lax.axis_index("c")
    c0w = wid * A_PER_W

    def _off(c):
        return pl.multiple_of(c * 128, 128)

    def start_in(c, b):
        pltpu.async_copy(
            tt_hbm.at[:, pl.ds(_off(c), 128)], in_v.at[b], sem_i[b])

    def wait_in(c, b):
        pltpu.make_async_copy(
            tt_hbm.at[:, pl.ds(_off(c), 128)], in_v.at[b], sem_i[b]).wait()

    def start_out(c, b):
        pltpu.async_copy(
            out_v.at[b], scr_hbm.at[pl.ds(_off(c), 128)], sem_o[b])

    def wait_out(c, b):
        pltpu.make_async_copy(
            out_v.at[b], scr_hbm.at[pl.ds(_off(c), 128)], sem_o[b]).wait()

    iota, perms = _perms()
    start_in(c0w, 0)

    def body(k, carry):
        for b in range(2):
            j = k * 2 + b
            c = c0w + j
            wait_in(c, b)
            start_in(c + 1, 1 - b)

            @pl.when(k > 0)
            def _():
                wait_out(c - 2, b)

            _transpose_dmajor_to_vmajor(in_v.at[b], out_v.at[b], iota, perms, 8)
            start_out(c, b)
        return carry

    lax.fori_loop(0, A_PER_W // 2, body, 0)
    wait_in(c0w + A_PER_W, 0)
    wait_out(c0w + A_PER_W - 2, 0)
    wait_out(c0w + A_PER_W - 1, 1)

    @pl.when(wid < A_EXTRA)
    def _():
        c = A_FULL + wid
        pltpu.sync_copy(tt_hbm.at[:, pl.ds(c * 128, 128)], in_v.at[0])
        _transpose_dmajor_to_vmajor(in_v.at[0], out_v.at[0], iota, perms, 8)
        pltpu.sync_copy(out_v.at[0], scr_hbm.at[pl.ds(c * 128, 128)])

    @pl.when(wid == A_EXTRA)
    def _():
        pltpu.sync_copy(tail_hbm, out_v.at[0])
        pltpu.sync_copy(out_v.at[0], scr_hbm.at[pl.ds(7812 * 128, 128)])


@functools.partial(
    pl.kernel,
    mesh=mesh,
    out_type=jax.ShapeDtypeStruct((HIST, D_MODEL, BATCH), jnp.float32),
    scratch_types=[
        pltpu.VMEM((2, 8, 128), jnp.int32),
        pltpu.VMEM((2, 128, 128), jnp.float32),
        pltpu.VMEM((2, D_MODEL, 128), jnp.float32),
        [pltpu.SemaphoreType.DMA] * 2,
        [pltpu.SemaphoreType.DMA] * 2,
        [pltpu.SemaphoreType.DMA] * 2,
    ],
    compiler_params=_params,
)
def _gather_transpose(xt_hbm, scr_hbm, out_hbm, idx_v, g_v, t_v,
                      sem_x, sem_g, sem_o):
    wid = lax.axis_index("s") * NC + lax.axis_index("c")
    n0 = wid * B_BLK_PER_W

    def hb(n):
        h0 = pl.multiple_of(lax.shift_left(lax.shift_right_logical(n, 5), 3), 8)
        b0 = pl.multiple_of(lax.shift_left(n & 31, 7), 128)
        return h0, b0

    def start_idx(n, u):
        h0, b0 = hb(n)
        pltpu.async_copy(
            xt_hbm.at[pl.ds(h0, 8), pl.ds(b0, 128)], idx_v.at[u], sem_x[u])

    def wait_idx(n, u):
        h0, b0 = hb(n)
        pltpu.make_async_copy(
            xt_hbm.at[pl.ds(h0, 8), pl.ds(b0, 128)], idx_v.at[u],
            sem_x[u]).wait()

    def start_gather(u, s, b):
        pltpu.async_copy(scr_hbm.at[idx_v.at[u].at[s]], g_v.at[b], sem_g[b])

    def wait_gather(u, s, b):
        pltpu.make_async_copy(
            scr_hbm.at[idx_v.at[u].at[s]], g_v.at[b], sem_g[b]).wait()

    def start_out(n, s, b):
        h0, b0 = hb(n)
        pltpu.async_copy(
            t_v.at[b], out_hbm.at[h0 + s, :, pl.ds(b0, 128)], sem_o[b])

    def wait_out(n, s, b):
        h0, b0 = hb(n)
        pltpu.make_async_copy(
            t_v.at[b], out_hbm.at[h0 + s, :, pl.ds(b0, 128)], sem_o[b]).wait()

    def block(n, u, prefetch_next):
        start_gather(u, 0, 0)
        if prefetch_next:
            start_idx(n + 1, 1 - u)

        def sub(k2, carry):
            for b in range(2):
                s = k2 * 2 + b
                if b == 0:
                    start_gather(u, s + 1, 1)
                else:
                    @pl.when(k2 < 3)
                    def _():
                        start_gather(u, s + 1, 0)
                wait_gather(u, s, b)

                @pl.when(k2 > 0)
                def _():
                    wait_out(n, s - 2, b)

                _transpose_vmajor_to_dmajor(g_v.at[b], t_v.at[b], iota, perms)
                start_out(n, s, b)
            return carry

        lax.fori_loop(0, 4, sub, 0)
        wait_out(n, 6, 0)
        wait_out(n, 7, 1)

    iota, perms = _perms()
    start_idx(n0, 0)

    def body(k, carry):
        for u in range(2):
            n = n0 + k * 2 + u
            wait_idx(n, u)
            block(n, u, prefetch_next=True)
        return carry

    lax.fori_loop(0, B_BLK_PER_W // 2, body, 0)
    n = n0 + B_BLK_PER_W - 1
    wait_idx(n, 0)
    block(n, 0, prefetch_next=False)


def kernel(x, table):
    xt = x.T
    tt = table.T
    tail = jnp.pad(table[VOCAB - 64:], ((0, 64), (0, 64)))
    scratch = _table_transpose(tt, tail)
    out_t = _gather_transpose(xt, scratch)
    return jnp.transpose(out_t, (2, 0, 1))


# speedup vs baseline: 2.3553x; 1.0706x over previous
import functools

import jax
import jax.numpy as jnp
from jax import lax
from jax.experimental import pallas as pl
from jax.experimental.pallas import tpu as pltpu
from jax.experimental.pallas import tpu_sc as plsc

VOCAB = 1000000
D_MODEL = 64
BATCH = 4096
HIST = 200

_info = plsc.get_sparse_core_info()
NC, NS = _info.num_cores, _info.num_subcores
NW = NC * NS

NBLK = BATCH // 128
HGRP = HIST // 8
B_BLOCKS = HGRP * NBLK
B_BLK_PER_W = B_BLOCKS // NW

mesh = plsc.VectorSubcoreMesh(core_axis_name="c", subcore_axis_name="s")
_params = pltpu.CompilerParams(
    use_tc_tiling_on_sc=True, needs_layout_passes=False)


def _perms():
    iota = lax.iota(jnp.int32, 16)
    return iota, [(iota + k) & 15 for k in range(16)]


def _transpose_vmajor_to_dmajor(src, dst, iota, dcols):
    def body(bg, carry):
        brows = bg * 16 + iota
        for dcol in dcols:
            vals = plsc.load_gather(src, [brows, dcol])
            plsc.store_scatter(dst, [dcol, brows], vals)
        return carry

    lax.fori_loop(0, 8, body, 0)


@functools.partial(
    pl.kernel,
    mesh=mesh,
    out_type=jax.ShapeDtypeStruct((HIST, D_MODEL, BATCH), jnp.float32),
    scratch_types=[
        pltpu.VMEM((2, 8, 128), jnp.int32),
        pltpu.VMEM((2, 128, 128), jnp.float32),
        pltpu.VMEM((2, D_MODEL, 128), jnp.float32),
        [pltpu.SemaphoreType.DMA] * 2,
        [pltpu.SemaphoreType.DMA] * 2,
        [pltpu.SemaphoreType.DMA] * 2,
    ],
    compiler_params=_params,
)
def _gather_transpose(xt_hbm, tab_hbm, out_hbm, idx_v, g_v, t_v,
                      sem_x, sem_g, sem_o):
    wid = lax.axis_index("s") * NC + lax.axis_index("c")
    n0 = wid * B_BLK_PER_W

    def hb(n):
        h0 = pl.multiple_of(lax.shift_left(lax.shift_right_logical(n, 5), 3), 8)
        b0 = pl.multiple_of(lax.shift_left(n & 31, 7), 128)
        return h0, b0

    def start_idx(n, u):
        h0, b0 = hb(n)
        pltpu.async_copy(
            xt_hbm.at[pl.ds(h0, 8), pl.ds(b0, 128)], idx_v.at[u], sem_x[u])

    def wait_idx(n, u):
        h0, b0 = hb(n)
        pltpu.make_async_copy(
            xt_hbm.at[pl.ds(h0, 8), pl.ds(b0, 128)], idx_v.at[u],
            sem_x[u]).wait()

    def start_gather(u, s, b):
        pltpu.async_copy(tab_hbm.at[idx_v.at[u].at[s]], g_v.at[b], sem_g[b])

    def wait_gather(u, s, b):
        pltpu.make_async_copy(
            tab_hbm.at[idx_v.at[u].at[s]], g_v.at[b], sem_g[b]).wait()

    def start_out(n, s, b):
        h0, b0 = hb(n)
        pltpu.async_copy(
            t_v.at[b], out_hbm.at[h0 + s, :, pl.ds(b0, 128)], sem_o[b])

    def wait_out(n, s, b):
        h0, b0 = hb(n)
        pltpu.make_async_copy(
            t_v.at[b], out_hbm.at[h0 + s, :, pl.ds(b0, 128)], sem_o[b]).wait()

    def block(n, u, prefetch_next):
        start_gather(u, 0, 0)
        if prefetch_next:
            start_idx(n + 1, 1 - u)

        def sub(k2, carry):
            for b in range(2):
                s = k2 * 2 + b
                if b == 0:
                    start_gather(u, s + 1, 1)
                else:
                    @pl.when(k2 < 3)
                    def _():
                        start_gather(u, s + 1, 0)
                wait_gather(u, s, b)

                @pl.when(k2 > 0)
                def _():
                    wait_out(n, s - 2, b)

                _transpose_vmajor_to_dmajor(g_v.at[b], t_v.at[b], iota, dcols)
                start_out(n, s, b)
            return carry

        lax.fori_loop(0, 4, sub, 0)
        wait_out(n, 6, 0)
        wait_out(n, 7, 1)

    iota, perms = _perms()
    dcols = [p + dg * 16 for dg in range(D_MODEL // 16) for p in perms]
    start_idx(n0, 0)

    def body(k, carry):
        for u in range(2):
            n = n0 + k * 2 + u
            wait_idx(n, u)
            block(n, u, prefetch_next=True)
        return carry

    lax.fori_loop(0, B_BLK_PER_W // 2, body, 0)
    n = n0 + B_BLK_PER_W - 1
    wait_idx(n, 0)
    block(n, 0, prefetch_next=False)


def kernel(x, table):
    xt = x.T
    tab128 = jnp.pad(table, ((0, 0), (0, D_MODEL)))
    out_t = _gather_transpose(xt, tab128)
    return jnp.transpose(out_t, (2, 0, 1))
